# baseline (speedup 1.0000x reference)
import functools

import jax
import jax.numpy as jnp
from jax import lax
from jax.experimental import pallas as pl
from jax.experimental.pallas import tpu as pltpu
from jax.experimental.pallas import tpu_sc as plsc

NC = 2
NS = 16
NW = NC * NS
C = 64
L = 16
PD = 4


def _sc_aggregate(feature, packed_w, n_chunks, n_pad, D):
    mesh = plsc.VectorSubcoreMesh(
        core_axis_name="c", subcore_axis_name="s", num_cores=NC, num_subcores=NS
    )
    rows_per = n_pad // NS
    zeros = jnp.zeros((rows_per, D), jnp.float32)

    idx_types = [pltpu.VMEM((C,), jnp.int32) for _ in range(2 * PD)]
    row_types = [pltpu.VMEM((C, D), jnp.float32) for _ in range(PD)]
    sem_types = [pltpu.SemaphoreType.DMA for _ in range(2 * PD)]

    @functools.partial(
        pl.kernel,
        mesh=mesh,
        out_type=jax.ShapeDtypeStruct((NC, n_pad, D), jnp.float32),
        scratch_types=[pltpu.VMEM((n_chunks * C,), jnp.int32)]
        + idx_types + row_types
        + [pltpu.VMEM_SHARED((n_pad, D), jnp.float32)]
        + sem_types,
    )
    def agg_kernel(packed_hbm, z_hbm, feat_hbm, out_hbm, packed_v, *rest):
        s_idx = rest[0:2 * PD:2]
        d_idx = rest[1:2 * PD:2]
        rows = rest[2 * PD:3 * PD]
        agg_sh = rest[3 * PD]
        sem_g = rest[3 * PD + 1:3 * PD + 1 + PD]
        sem_s = rest[3 * PD + 1 + PD:3 * PD + 1 + 2 * PD]

        c = lax.axis_index("c")
        s = lax.axis_index("s")
        wid = s * NC + c
        row0 = s * rows_per
        pltpu.sync_copy(z_hbm, agg_sh.at[pl.ds(row0, rows_per)])
        pltpu.sync_copy(packed_hbm.at[wid], packed_v)
        plsc.subcore_barrier()

        def unpack(j, s_ref, d_ref):
            for g in range(C // L):
                v = packed_v[pl.ds(j * C + g * L, L)]
                s_ref[pl.ds(g * L, L)] = v & 0xFFFF
                d_ref[pl.ds(g * L, L)] = lax.shift_right_logical(v, 16)

        unpack(0, s_idx[0], d_idx[0])
        pltpu.sync_copy(feat_hbm.at[s_idx[0]], rows[0])

        @pl.loop(0, n_chunks, step=PD)
        def _(j0):
            gs = [None] * PD
            scs = [None] * PD
            for b in range(1, PD):
                unpack(j0 + b, s_idx[b], d_idx[b])
                gs[b] = pltpu.async_copy(feat_hbm.at[s_idx[b]], rows[b],
                                         sem_g[b])
            scs[0] = pltpu.async_copy(rows[0], agg_sh.at[d_idx[0]],
                                      sem_s[0], add=True)
            for b in range(1, PD):
                gs[b].wait()
                scs[b] = pltpu.async_copy(rows[b], agg_sh.at[d_idx[b]],
                                          sem_s[b], add=True)
            scs[0].wait()
            unpack(jnp.minimum(j0 + PD, n_chunks - 1), s_idx[0], d_idx[0])
            g0 = pltpu.async_copy(feat_hbm.at[s_idx[0]], rows[0], sem_g[0])
            for b in range(1, PD):
                scs[b].wait()
            g0.wait()

        plsc.subcore_barrier()
        pltpu.sync_copy(agg_sh.at[pl.ds(row0, rows_per)],
                        out_hbm.at[c, pl.ds(row0, rows_per)])

    return agg_kernel(packed_w, zeros, feature)


def _tc_dense(feature, parts, W1, b1, W2, b2, gamma, beta, N, D):
    def body(f_ref, p_ref, w1_ref, b1_ref, w2_ref, b2_ref, g_ref, bt_ref,
             o_ref):
        h = f_ref[...] + p_ref[0, :N, :] + p_ref[1, :N, :]
        h = lax.dot_general(h, w1_ref[...], (((1,), (1,)), ((), ())),
                            preferred_element_type=jnp.float32) + b1_ref[...]
        h = jnp.maximum(h, 0.0)
        h = lax.dot_general(h, w2_ref[...], (((1,), (1,)), ((), ())),
                            preferred_element_type=jnp.float32) + b2_ref[...]
        h = jnp.maximum(h, 0.0)
        mean = jnp.mean(h, axis=0, keepdims=True)
        cent = h - mean
        var = jnp.mean(cent * cent, axis=0, keepdims=True)
        o_ref[...] = (g_ref[...] * cent * lax.rsqrt(var + 1e-5) + bt_ref[...])

    return pl.pallas_call(
        body,
        out_shape=jax.ShapeDtypeStruct((N, D), jnp.float32),
    )(feature, parts, W1, b1.reshape(1, D), W2, b2.reshape(1, D),
      gamma.reshape(1, D), beta.reshape(1, D))


def kernel(feature, edge_index, W1, b1, W2, b2, gamma, beta):
    N, D = feature.shape
    E = edge_index.shape[1]
    per = E // NW
    n_chunks = -(-per // C)
    n_chunks = -(-n_chunks // PD) * PD
    per_pad = n_chunks * C
    pad_cnt = per_pad - per
    n_pad = -(-(N + 1) // (NS * 8)) * (NS * 8)
    pad_rows = n_pad - N

    ei = edge_index.astype(jnp.int32)
    src = ei[0].reshape(NW, per)
    dst = ei[1].reshape(NW, per)
    pad_iota = jnp.arange(NW * pad_cnt, dtype=jnp.int32).reshape(NW, pad_cnt)
    pad_src = pad_iota % N
    pad_dst = N + pad_iota % pad_rows
    src_w = jnp.concatenate([src, pad_src], axis=1)
    dst_w = jnp.concatenate([dst, pad_dst], axis=1)
    packed_w = (src_w | (dst_w << 16)).reshape(NW, per_pad)

    parts = _sc_aggregate(feature, packed_w, n_chunks, n_pad, D)
    return _tc_dense(feature, parts, W1, b1, W2, b2, gamma, beta, N, D)
